# baseline (speedup 1.0000x reference)
import functools

import jax
import jax.numpy as jnp
from jax import lax
from jax.experimental import pallas as pl
from jax.experimental.pallas import tpu as pltpu
from jax.experimental.pallas import tpu_sc as plsc

B = 16
C = 3
N = 4096
K = 8
CN = C * N
HALF = CN // 2
P = 64
NCODE = K * K
TBL = 80
LANES = 16


def _tc_quantize(x, levels):
    def body(x_ref, lev_ref, out_ref):
        for c in range(C):
            v = x_ref[:, c, :, :]
            bestd = jnp.abs(v - lev_ref[c, 0])
            best = jnp.zeros_like(v, dtype=jnp.int32)
            for m in range(1, K):
                dm = jnp.abs(v - lev_ref[c, m])
                take = dm < bestd
                best = jnp.where(take, m, best)
                bestd = jnp.where(take, dm, bestd)
            out_ref[:, c, :, :] = best

    return pl.pallas_call(
        body,
        in_specs=[
            pl.BlockSpec(memory_space=pltpu.VMEM),
            pl.BlockSpec(memory_space=pltpu.SMEM),
        ],
        out_specs=pl.BlockSpec(memory_space=pltpu.VMEM),
        out_shape=jax.ShapeDtypeStruct(x.shape, jnp.int32),
    )(x, levels)


def _sc_presence(sym, iext, pmap_flat):
    mesh = plsc.VectorSubcoreMesh(core_axis_name="c", subcore_axis_name="s")
    out_type = jax.ShapeDtypeStruct((2 * B + P, TBL), jnp.int32)

    @functools.partial(
        pl.kernel,
        mesh=mesh,
        out_type=out_type,
        compiler_params=pltpu.CompilerParams(
            needs_layout_passes=False, use_tc_tiling_on_sc=False),
        scratch_types=[
            pltpu.VMEM((CN,), jnp.int32),
            pltpu.VMEM((CN + LANES,), jnp.int32),
            pltpu.VMEM((N + LANES,), jnp.int32),
            pltpu.VMEM((N + LANES,), jnp.int32),
            pltpu.VMEM((TBL,), jnp.int32),
            pltpu.VMEM((2, TBL), jnp.int32),
            pltpu.SemaphoreType.DMA,
            pltpu.SemaphoreType.DMA,
            pltpu.SemaphoreType.DMA,
            pltpu.SemaphoreType.DMA,
        ],
    )
    def k(sym_hbm, iext_hbm, pmap_hbm, pres_hbm,
          sym_v, iext_v, prow_a, prow_b, btab_v, ptab_v,
          sem0, sem1, sem2, sem3):
        wid = lax.axis_index("s") * 2 + lax.axis_index("c")
        half = (wid >= B).astype(jnp.int32)
        b = wid - half * B
        iota = lax.iota(jnp.int32, LANES)
        zero = iota * 0
        one = zero + 1

        cp_pa = pltpu.async_copy(pmap_hbm.at[2 * wid], prow_a.at[pl.ds(0, N)],
                                 sem0)
        cp_pb = pltpu.async_copy(pmap_hbm.at[2 * wid + 1],
                                 prow_b.at[pl.ds(0, N)], sem1)
        cp_sym = pltpu.async_copy(sym_hbm.at[b], sym_v, sem2)
        cp_iext = pltpu.async_copy(iext_hbm, iext_v, sem3)

        for t in range(TBL // LANES):
            btab_v[pl.ds(t * LANES, LANES)] = zero
            ptab_v[0, pl.ds(t * LANES, LANES)] = zero
            ptab_v[1, pl.ds(t * LANES, LANES)] = zero

        cp_pa.wait()
        cp_pb.wait()
        for r, prow_v in ((0, prow_a), (1, prow_b)):
            @plsc.parallel_loop(0, N - LANES, step=LANES, unroll=8)
            def pbody(base, r=r, prow_v=prow_v):
                a = prow_v[pl.ds(base, LANES)]
                bb = prow_v[pl.ds(base + 1, LANES)]
                plsc.store_scatter(ptab_v.at[r], [a * K + bb], one)

            base = N - LANES
            a = prow_v[pl.ds(base, LANES)]
            bb = prow_v[pl.ds(base + 1, LANES)]
            code = jnp.where(iota < LANES - 1, a * K + bb, NCODE)
            plsc.store_scatter(ptab_v.at[r], [code], one)

        out_p = pltpu.async_copy(
            ptab_v, pres_hbm.at[pl.ds(2 * B + 2 * wid, 2)], sem0)

        cp_sym.wait()
        cp_iext.wait()
        h_off = half * HALF

        @plsc.parallel_loop(0, HALF - LANES, step=LANES, unroll=8)
        def sbody(j):
            base = h_off + j
            ia = iext_v[pl.ds(base, LANES)]
            ib = iext_v[pl.ds(base + 1, LANES)]
            ga = plsc.load_gather(sym_v, [ia])
            gb = plsc.load_gather(sym_v, [ib])
            plsc.store_scatter(btab_v, [ga * K + gb], one)

        base = h_off + HALF - LANES
        ia = iext_v[pl.ds(base, LANES)]
        ib = iext_v[pl.ds(base + 1, LANES)]
        ga = plsc.load_gather(sym_v, [ia])
        gb = plsc.load_gather(sym_v, [ib])
        code = ga * K + gb
        code = jnp.where((half == 0) | (iota < LANES - 1), code, NCODE)
        plsc.store_scatter(btab_v, [code], one)

        @pl.when(half == 1)
        def _stage():
            btab_v[pl.ds(NCODE, LANES)] = ga

        out_s = pltpu.async_copy(btab_v, pres_hbm.at[wid], sem1)
        out_p.wait()
        out_s.wait()

    return k(sym, iext, pmap_flat)


def _tc_combine(pres, pf):
    def body(pres_ref, pf_ref, out_ref):
        psh = pres_ref[0:2 * B, :]
        psv = jnp.maximum(psh[0:B, 0:NCODE],
                          psh[B:2 * B, 0:NCODE]).astype(jnp.float32)
        ppv = pres_ref[2 * B:, 0:NCODE].astype(jnp.float32)
        sl = psh[B:2 * B, TBL - 1:TBL]
        pf = pf_ref[...]
        cs = jnp.sum(psv, axis=1, keepdims=True)
        cp_col = jnp.sum(ppv, axis=1, keepdims=True)
        ones_b = jnp.ones((B, 1), jnp.float32)
        cp = lax.dot_general(ones_b, cp_col, (((1,), (1,)), ((), ())))
        inter = lax.dot_general(psv, ppv, (((1,), (1,)), ((), ())))

        el = (sl == lax.broadcasted_iota(jnp.int32, (B, K), 1))
        el = el.astype(jnp.float32)
        ef = (pf == lax.broadcasted_iota(jnp.int32, (P, K), 1))
        ef = ef.astype(jnp.float32)

        a_mat = el[:, 0:1] * psv[:, 0:K]
        for a in range(1, K):
            a_mat = a_mat + el[:, a:a + 1] * psv[:, a * K:(a + 1) * K]
        a_at = lax.dot_general(a_mat, ef, (((1,), (1,)), ((), ())))

        bp_cols = [jnp.sum(ef * ppv[:, l * K:(l + 1) * K], axis=1,
                           keepdims=True) for l in range(K)]
        bp = jnp.concatenate(bp_cols, axis=1)
        b_at = lax.dot_general(el, bp, (((1,), (1,)), ((), ())))

        uj = jnp.maximum(a_at, b_at)
        csp = cs + cp - inter + (1.0 - uj)
        cmin = jnp.minimum(cs, cp)
        cmax = jnp.maximum(cs, cp)
        out_ref[...] = (csp - cmin) / cmax

    return pl.pallas_call(
        body,
        out_shape=jax.ShapeDtypeStruct((B, P), jnp.float32),
    )(pres, pf)


def kernel(x, curve, levels, pmap):
    curve = curve.astype(jnp.int32)
    ch_off = (jnp.arange(C, dtype=jnp.int32) * N)[:, None]
    idx = (curve[None, :] + ch_off).reshape(-1)
    iext = jnp.concatenate([idx, idx[-LANES:]])
    pmap_flat = pmap.reshape(P, N).astype(jnp.int32)

    sym = _tc_quantize(x, levels).reshape(B, CN)
    pres = _sc_presence(sym, iext, pmap_flat)
    pf = pmap_flat[:, 0:1]
    return _tc_combine(pres, pf)


# speedup vs baseline: 1.0061x; 1.0061x over previous
import functools

import jax
import jax.numpy as jnp
from jax import lax
from jax.experimental import pallas as pl
from jax.experimental.pallas import tpu as pltpu
from jax.experimental.pallas import tpu_sc as plsc

B = 16
C = 3
N = 4096
K = 8
CN = C * N
HALF = CN // 2
P = 64
NCODE = K * K
TBL = 80
LANES = 16


def _tc_quantize(x, levels):
    def body(x_ref, lev_ref, out_ref):
        for c in range(C):
            v = x_ref[:, c, :, :]
            bestd = jnp.abs(v - lev_ref[c, 0])
            best = jnp.zeros_like(v, dtype=jnp.int32)
            for m in range(1, K):
                dm = jnp.abs(v - lev_ref[c, m])
                take = dm < bestd
                best = jnp.where(take, m, best)
                bestd = jnp.where(take, dm, bestd)
            out_ref[:, c, :, :] = best

    return pl.pallas_call(
        body,
        in_specs=[
            pl.BlockSpec(memory_space=pltpu.VMEM),
            pl.BlockSpec(memory_space=pltpu.SMEM),
        ],
        out_specs=pl.BlockSpec(memory_space=pltpu.VMEM),
        out_shape=jax.ShapeDtypeStruct(x.shape, jnp.int32),
    )(x, levels)


def _sc_presence(sym, iext, pmap_flat):
    mesh = plsc.VectorSubcoreMesh(core_axis_name="c", subcore_axis_name="s")
    out_type = jax.ShapeDtypeStruct((2 * B + P, TBL), jnp.int32)

    @functools.partial(
        pl.kernel,
        mesh=mesh,
        out_type=out_type,
        compiler_params=pltpu.CompilerParams(
            needs_layout_passes=False, use_tc_tiling_on_sc=False),
        scratch_types=[
            pltpu.VMEM((CN,), jnp.int32),
            pltpu.VMEM((CN + LANES,), jnp.int32),
            pltpu.VMEM((N + LANES,), jnp.int32),
            pltpu.VMEM((N + LANES,), jnp.int32),
            pltpu.VMEM((TBL,), jnp.int32),
            pltpu.VMEM((2, TBL), jnp.int32),
            pltpu.SemaphoreType.DMA,
            pltpu.SemaphoreType.DMA,
            pltpu.SemaphoreType.DMA,
            pltpu.SemaphoreType.DMA,
        ],
    )
    def k(sym_hbm, iext_hbm, pmap_hbm, pres_hbm,
          sym_v, iext_v, prow_a, prow_b, btab_v, ptab_v,
          sem0, sem1, sem2, sem3):
        wid = lax.axis_index("s") * 2 + lax.axis_index("c")
        half = (wid >= B).astype(jnp.int32)
        b = wid - half * B
        iota = lax.iota(jnp.int32, LANES)
        zero = iota * 0
        one = zero + 1

        cp_pa = pltpu.async_copy(pmap_hbm.at[2 * wid], prow_a.at[pl.ds(0, N)],
                                 sem0)
        cp_pb = pltpu.async_copy(pmap_hbm.at[2 * wid + 1],
                                 prow_b.at[pl.ds(0, N)], sem1)
        cp_sym = pltpu.async_copy(sym_hbm.at[b], sym_v, sem2)
        cp_iext = pltpu.async_copy(iext_hbm, iext_v, sem3)

        for t in range(TBL // LANES):
            btab_v[pl.ds(t * LANES, LANES)] = zero
            ptab_v[0, pl.ds(t * LANES, LANES)] = zero
            ptab_v[1, pl.ds(t * LANES, LANES)] = zero

        cp_pa.wait()
        cp_pb.wait()
        for r, prow_v in ((0, prow_a), (1, prow_b)):
            @plsc.parallel_loop(0, N - LANES, step=LANES, unroll=4)
            def pbody(base, r=r, prow_v=prow_v):
                a = prow_v[pl.ds(base, LANES)]
                bb = prow_v[pl.ds(base + 1, LANES)]
                plsc.store_scatter(ptab_v.at[r], [a * K + bb], one)

            base = N - LANES
            a = prow_v[pl.ds(base, LANES)]
            bb = prow_v[pl.ds(base + 1, LANES)]
            code = jnp.where(iota < LANES - 1, a * K + bb, NCODE)
            plsc.store_scatter(ptab_v.at[r], [code], one)

        out_p = pltpu.async_copy(
            ptab_v, pres_hbm.at[pl.ds(2 * B + 2 * wid, 2)], sem0)

        cp_sym.wait()
        cp_iext.wait()
        h_off = half * HALF

        @plsc.parallel_loop(0, HALF - LANES, step=LANES, unroll=4)
        def sbody(j):
            base = h_off + j
            ia = iext_v[pl.ds(base, LANES)]
            ib = iext_v[pl.ds(base + 1, LANES)]
            ga = plsc.load_gather(sym_v, [ia])
            gb = plsc.load_gather(sym_v, [ib])
            plsc.store_scatter(btab_v, [ga * K + gb], one)

        base = h_off + HALF - LANES
        ia = iext_v[pl.ds(base, LANES)]
        ib = iext_v[pl.ds(base + 1, LANES)]
        ga = plsc.load_gather(sym_v, [ia])
        gb = plsc.load_gather(sym_v, [ib])
        code = ga * K + gb
        code = jnp.where((half == 0) | (iota < LANES - 1), code, NCODE)
        plsc.store_scatter(btab_v, [code], one)

        @pl.when(half == 1)
        def _stage():
            btab_v[pl.ds(NCODE, LANES)] = ga

        out_s = pltpu.async_copy(btab_v, pres_hbm.at[wid], sem1)
        out_p.wait()
        out_s.wait()

    return k(sym, iext, pmap_flat)


def _tc_combine(pres, pf):
    def body(pres_ref, pf_ref, out_ref):
        psh = pres_ref[0:2 * B, :]
        psv = jnp.maximum(psh[0:B, 0:NCODE],
                          psh[B:2 * B, 0:NCODE]).astype(jnp.float32)
        ppv = pres_ref[2 * B:, 0:NCODE].astype(jnp.float32)
        sl = psh[B:2 * B, TBL - 1:TBL]
        pf = pf_ref[...]
        cs = jnp.sum(psv, axis=1, keepdims=True)
        cp_col = jnp.sum(ppv, axis=1, keepdims=True)
        ones_b = jnp.ones((B, 1), jnp.float32)
        cp = lax.dot_general(ones_b, cp_col, (((1,), (1,)), ((), ())))
        inter = lax.dot_general(psv, ppv, (((1,), (1,)), ((), ())))

        el = (sl == lax.broadcasted_iota(jnp.int32, (B, K), 1))
        el = el.astype(jnp.float32)
        ef = (pf == lax.broadcasted_iota(jnp.int32, (P, K), 1))
        ef = ef.astype(jnp.float32)

        a_mat = el[:, 0:1] * psv[:, 0:K]
        for a in range(1, K):
            a_mat = a_mat + el[:, a:a + 1] * psv[:, a * K:(a + 1) * K]
        a_at = lax.dot_general(a_mat, ef, (((1,), (1,)), ((), ())))

        bp_cols = [jnp.sum(ef * ppv[:, l * K:(l + 1) * K], axis=1,
                           keepdims=True) for l in range(K)]
        bp = jnp.concatenate(bp_cols, axis=1)
        b_at = lax.dot_general(el, bp, (((1,), (1,)), ((), ())))

        uj = jnp.maximum(a_at, b_at)
        csp = cs + cp - inter + (1.0 - uj)
        cmin = jnp.minimum(cs, cp)
        cmax = jnp.maximum(cs, cp)
        out_ref[...] = (csp - cmin) / cmax

    return pl.pallas_call(
        body,
        out_shape=jax.ShapeDtypeStruct((B, P), jnp.float32),
    )(pres, pf)


def kernel(x, curve, levels, pmap):
    curve = curve.astype(jnp.int32)
    ch_off = (jnp.arange(C, dtype=jnp.int32) * N)[:, None]
    idx = (curve[None, :] + ch_off).reshape(-1)
    iext = jnp.concatenate([idx, idx[-LANES:]])
    pmap_flat = pmap.reshape(P, N).astype(jnp.int32)

    sym = _tc_quantize(x, levels).reshape(B, CN)
    pres = _sc_presence(sym, iext, pmap_flat)
    pf = pmap_flat[:, 0:1]
    return _tc_combine(pres, pf)
